# Initial kernel scaffold; baseline (speedup 1.0000x reference)
import jax
import jax.numpy as jnp
from jax.experimental import pallas as pl


def kernel(x, edge_index, batch, W1, b1, W2, b2, W3, b3, W4, b4, W5, b5, Wm1, bm1, Wm2, bm2):
    raise NotImplementedError("write your pallas kernel here")


# speedup vs baseline: 10.9150x; 10.9150x over previous
import functools

import jax
import jax.numpy as jnp
from jax import lax
from jax.experimental import pallas as pl
from jax.experimental.pallas import tpu as pltpu
from jax.experimental.pallas import tpu_sc as plsc

N = 10000
E = 320000
NC = 2
NS = 16
CH = 128
NCH = E // CH
RPB = 624
F = 128


def _mesh():
    return plsc.VectorSubcoreMesh(core_axis_name="c", subcore_axis_name="s")


def _degree_call(dst2d):
    @functools.partial(
        pl.kernel,
        mesh=_mesh(),
        out_type=jax.ShapeDtypeStruct((NC * NS, N), jnp.float32),
        compiler_params=pltpu.CompilerParams(needs_layout_passes=False),
        scratch_types=[
            pltpu.VMEM((N,), jnp.float32),
            pltpu.VMEM((CH,), jnp.int32),
        ],
    )
    def k(dst_hbm, out_hbm, hist_v, idx_v):
        c = lax.axis_index("c")
        s = lax.axis_index("s")
        tid = s * NC + c
        z16 = jnp.zeros((16,), jnp.float32)

        def zero_body(i, carry):
            hist_v[pl.ds(i * 16, 16)] = z16
            return carry

        lax.fori_loop(0, N // 16, zero_body, 0)

        ones = jnp.ones((16,), jnp.float32)
        nloop = (NCH + NC * NS - 1) // (NC * NS)

        def chunk_body(kk, carry):
            j = kk * (NC * NS) + tid

            @pl.when(j < NCH)
            def _():
                pltpu.sync_copy(dst_hbm.at[j], idx_v)

                def inner(i, c2):
                    idx = idx_v[pl.ds(i * 16, 16)]
                    plsc.addupdate_scatter(hist_v, [idx], ones)
                    return c2

                lax.fori_loop(0, CH // 16, inner, 0)

            return carry

        lax.fori_loop(0, nloop, chunk_body, 0)
        pltpu.sync_copy(hist_v, out_hbm.at[tid])

    return k(dst2d)


def _make_prop(feat_split):
    @functools.partial(
        pl.kernel,
        mesh=_mesh(),
        out_type=jax.ShapeDtypeStruct((NC, N, F), jnp.float32),
        scratch_types=[
            pltpu.VMEM((CH,), jnp.int32),
            pltpu.VMEM((1, CH), jnp.int32),
            pltpu.VMEM((CH, F), jnp.float32),
            pltpu.VMEM((208, F), jnp.float32),
            pltpu.VMEM_SHARED((N, F), jnp.float32),
            pltpu.SemaphoreType.DMA,
        ],
    )
    def k(src_hbm, dst_hbm, g_hbm, out_hbm, src_v, dst_v, rows_v, zero_v,
          slab, sem):
        c = lax.axis_index("c")
        s = lax.axis_index("s")
        z16 = jnp.zeros((16,), jnp.float32)

        def zb(i, carry):
            zero_v[i // (F // 16), pl.ds((i % (F // 16)) * 16, 16)] = z16
            return carry

        lax.fori_loop(0, 208 * (F // 16), zb, 0)
        for q in range(RPB // 208):
            pltpu.sync_copy(zero_v, slab.at[pl.ds(s * RPB + q * 208, 208)])

        @pl.when(s == 0)
        def _zero_tail():
            pltpu.sync_copy(zero_v.at[pl.ds(0, N - NS * RPB)],
                            slab.at[pl.ds(NS * RPB, N - NS * RPB)])

        plsc.subcore_barrier()

        if feat_split:
            stride = NS
            base = s
        else:
            stride = NC * NS
            base = s * NC + c
        nloop = (NCH + stride - 1) // stride

        def body(kk, carry):
            j = kk * stride + base

            @pl.when(j < NCH)
            def _():
                pltpu.sync_copy(src_hbm.at[j], src_v)
                pltpu.sync_copy(dst_hbm.at[j], dst_v.at[0])
                if feat_split:
                    off = c * N

                    def addoff(i, c2):
                        sl = pl.ds(i * 16, 16)
                        src_v[sl] = src_v[sl] + off
                        return c2

                    lax.fori_loop(0, CH // 16, addoff, 0)
                pltpu.async_copy(g_hbm.at[src_v], rows_v, sem).wait()
                pltpu.sync_copy(rows_v, slab.at[dst_v.at[0]], add=True)

            return carry

        lax.fori_loop(0, nloop, body, 0)
        plsc.subcore_barrier()
        pltpu.sync_copy(slab.at[pl.ds(s * RPB, RPB)],
                        out_hbm.at[c, pl.ds(s * RPB, RPB)])

        @pl.when(s == 0)
        def _drain_tail():
            pltpu.sync_copy(slab.at[pl.ds(NS * RPB, N - NS * RPB)],
                            out_hbm.at[c, pl.ds(NS * RPB, N - NS * RPB)])

    return k


_PROP_CACHE = {}


def _prop_feat(src2d, dst2d, g):
    if True not in _PROP_CACHE:
        _PROP_CACHE[True] = _make_prop(True)
    return _PROP_CACHE[True](src2d, dst2d, g)


def _prop_edge(src2d, dst2d, g):
    if False not in _PROP_CACHE:
        _PROP_CACHE[False] = _make_prop(False)
    return _PROP_CACHE[False](src2d, dst2d, g)


_BR = 2000


def _dinv_call(partials):
    def body(p_ref, o_ref):
        deg = jnp.sum(p_ref[...], axis=0) + 1.0
        o_ref[...] = lax.rsqrt(deg)[:, None]

    return pl.pallas_call(
        body,
        out_shape=jax.ShapeDtypeStruct((N, 1), jnp.float32),
    )(partials)


def _lin1_call(x, W1, dinv):
    def body(x_ref, w_ref, d_ref, o_ref):
        o_ref[...] = (d_ref[...] * jnp.dot(
            x_ref[...], w_ref[...], preferred_element_type=jnp.float32))[None]

    return pl.pallas_call(
        body,
        grid=(NC, N // _BR),
        in_specs=[
            pl.BlockSpec((_BR, 128), lambda c, i: (i, 0)),
            pl.BlockSpec((128, F), lambda c, i: (0, c)),
            pl.BlockSpec((_BR, 1), lambda c, i: (i, 0)),
        ],
        out_specs=pl.BlockSpec((1, _BR, F), lambda c, i: (c, i, 0)),
        out_shape=jax.ShapeDtypeStruct((NC, N, F), jnp.float32),
    )(x, W1, dinv)


def _lin2_call(A1, g1, dinv, b1r, W2r):
    def body(a_ref, g_ref, d_ref, b_ref, w_ref, o_ref):
        d = d_ref[...]
        u0 = jnp.maximum(d * (a_ref[0] + g_ref[0]) + b_ref[0], 0.0)
        u1 = jnp.maximum(d * (a_ref[1] + g_ref[1]) + b_ref[1], 0.0)
        acc = jnp.dot(u0, w_ref[0], preferred_element_type=jnp.float32)
        acc = acc + jnp.dot(u1, w_ref[1], preferred_element_type=jnp.float32)
        o_ref[...] = d * acc

    return pl.pallas_call(
        body,
        grid=(N // _BR,),
        in_specs=[
            pl.BlockSpec((NC, _BR, F), lambda i: (0, i, 0)),
            pl.BlockSpec((NC, _BR, F), lambda i: (0, i, 0)),
            pl.BlockSpec((_BR, 1), lambda i: (i, 0)),
            pl.BlockSpec((NC, 1, F), lambda i: (0, 0, 0)),
            pl.BlockSpec((NC, F, F), lambda i: (0, 0, 0)),
        ],
        out_specs=pl.BlockSpec((_BR, F), lambda i: (i, 0)),
        out_shape=jax.ShapeDtypeStruct((N, F), jnp.float32),
    )(A1, g1, dinv, b1r, W2r)


def _lin_mid_call(S, g, dinv, br, W):
    def body(s_ref, g_ref, d_ref, b_ref, w_ref, o_ref):
        d = d_ref[...]
        u = jnp.maximum(d * (s_ref[0] + s_ref[1] + g_ref[...]) + b_ref[...],
                        0.0)
        o_ref[...] = d * jnp.dot(u, w_ref[...],
                                 preferred_element_type=jnp.float32)

    return pl.pallas_call(
        body,
        grid=(N // _BR,),
        in_specs=[
            pl.BlockSpec((NC, _BR, F), lambda i: (0, i, 0)),
            pl.BlockSpec((_BR, F), lambda i: (i, 0)),
            pl.BlockSpec((_BR, 1), lambda i: (i, 0)),
            pl.BlockSpec((1, F), lambda i: (0, 0)),
            pl.BlockSpec((F, F), lambda i: (0, 0)),
        ],
        out_specs=pl.BlockSpec((_BR, F), lambda i: (i, 0)),
        out_shape=jax.ShapeDtypeStruct((N, F), jnp.float32),
    )(S, g, dinv, br, W)


def _head_call(S, g, dinv, b5r, Wm1p, bm1p, Wm2p, bm2p):
    def body(s_ref, g_ref, d_ref, b_ref, w1_ref, c1_ref, w2_ref, c2_ref,
             o_ref):
        d = d_ref[...]
        u = jnp.maximum(d * (s_ref[0] + s_ref[1] + g_ref[...]) + b_ref[...],
                        0.0)
        h2 = jnp.maximum(
            jnp.dot(u, w1_ref[...], preferred_element_type=jnp.float32)
            + c1_ref[...], 0.0)
        o_ref[...] = jnp.dot(
            h2, w2_ref[...], preferred_element_type=jnp.float32) + c2_ref[...]

    return pl.pallas_call(
        body,
        grid=(N // _BR,),
        in_specs=[
            pl.BlockSpec((NC, _BR, F), lambda i: (0, i, 0)),
            pl.BlockSpec((_BR, F), lambda i: (i, 0)),
            pl.BlockSpec((_BR, 1), lambda i: (i, 0)),
            pl.BlockSpec((1, F), lambda i: (0, 0)),
            pl.BlockSpec((F, F), lambda i: (0, 0)),
            pl.BlockSpec((1, F), lambda i: (0, 0)),
            pl.BlockSpec((F, F), lambda i: (0, 0)),
            pl.BlockSpec((1, F), lambda i: (0, 0)),
        ],
        out_specs=pl.BlockSpec((_BR, F), lambda i: (i, 0)),
        out_shape=jax.ShapeDtypeStruct((N, F), jnp.float32),
    )(S, g, dinv, b5r, Wm1p, bm1p, Wm2p, bm2p)


def kernel(x, edge_index, batch, W1, b1, W2, b2, W3, b3, W4, b4, W5, b5,
           Wm1, bm1, Wm2, bm2):
    del batch
    src2d = edge_index[0].reshape(NCH, CH)
    dst2d = edge_index[1].reshape(NCH, CH)

    partials = _degree_call(dst2d)
    dinv = _dinv_call(partials)

    g1 = _lin1_call(x, W1, dinv)
    A1 = _prop_feat(src2d, dst2d, g1.reshape(NC * N, F))
    g2 = _lin2_call(A1, g1, dinv, b1.reshape(NC, 1, F),
                    W2.reshape(NC, F, F))

    g = g2
    for (bl, Wn) in ((b2, W3), (b3, W4), (b4, W5)):
        S = _prop_edge(src2d, dst2d, g)
        g = _lin_mid_call(S, g, dinv, bl.reshape(1, F), Wn)

    S5 = _prop_edge(src2d, dst2d, g)

    hm = Wm1.shape[1]
    Wm1p = jnp.pad(Wm1, ((0, 0), (0, F - hm)))
    bm1p = jnp.pad(bm1, (0, F - hm)).reshape(1, F)
    Wm2p = jnp.pad(Wm2, ((0, F - hm), (0, F - Wm2.shape[1])))
    bm2p = jnp.pad(bm2, (0, F - Wm2.shape[1])).reshape(1, F)

    out = _head_call(S5, g, dinv, b5.reshape(1, F), Wm1p, bm1p, Wm2p, bm2p)
    return out[:, :Wm2.shape[1]]


# speedup vs baseline: 15.6788x; 1.4364x over previous
import functools

import jax
import jax.numpy as jnp
from jax import lax
from jax.experimental import pallas as pl
from jax.experimental.pallas import tpu as pltpu
from jax.experimental.pallas import tpu_sc as plsc

N = 10000
E = 320000
NC = 2
NS = 16
CH = 128
NCH = E // CH
RPB = 624
F = 128


def _mesh():
    return plsc.VectorSubcoreMesh(core_axis_name="c", subcore_axis_name="s")


def _degree_call(dst2d):
    @functools.partial(
        pl.kernel,
        mesh=_mesh(),
        out_type=jax.ShapeDtypeStruct((NC * NS, N), jnp.float32),
        compiler_params=pltpu.CompilerParams(needs_layout_passes=False),
        scratch_types=[
            pltpu.VMEM((N,), jnp.float32),
            pltpu.VMEM((CH,), jnp.int32),
        ],
    )
    def k(dst_hbm, out_hbm, hist_v, idx_v):
        c = lax.axis_index("c")
        s = lax.axis_index("s")
        tid = s * NC + c
        z16 = jnp.zeros((16,), jnp.float32)

        def zero_body(i, carry):
            hist_v[pl.ds(i * 16, 16)] = z16
            return carry

        lax.fori_loop(0, N // 16, zero_body, 0)

        ones = jnp.ones((16,), jnp.float32)
        nloop = (NCH + NC * NS - 1) // (NC * NS)

        def chunk_body(kk, carry):
            j = kk * (NC * NS) + tid

            @pl.when(j < NCH)
            def _():
                pltpu.sync_copy(dst_hbm.at[j], idx_v)

                def inner(i, c2):
                    idx = idx_v[pl.ds(i * 16, 16)]
                    plsc.addupdate_scatter(hist_v, [idx], ones)
                    return c2

                lax.fori_loop(0, CH // 16, inner, 0)

            return carry

        lax.fori_loop(0, nloop, chunk_body, 0)
        pltpu.sync_copy(hist_v, out_hbm.at[tid])

    return k(dst2d)


def _make_prop(feat_split):
    R = 2

    @functools.partial(
        pl.kernel,
        mesh=_mesh(),
        out_type=jax.ShapeDtypeStruct((NC, N, F), jnp.float32),
        scratch_types=[
            pltpu.VMEM((R, CH), jnp.int32),
            pltpu.VMEM((R, CH), jnp.int32),
            pltpu.VMEM((R, CH, F), jnp.float32),
            pltpu.VMEM((16, F), jnp.float32),
            pltpu.VMEM_SHARED((N, F), jnp.float32),
            pltpu.SemaphoreType.DMA((R,)),
            pltpu.SemaphoreType.DMA,
            pltpu.SemaphoreType.DMA,
        ],
    )
    def k(src_hbm, dst_hbm, g_hbm, out_hbm, src_v, dst_v, rows_v, zero_v,
          slab, gsem, ssem, isem):
        c = lax.axis_index("c")
        s = lax.axis_index("s")
        z16 = jnp.zeros((16,), jnp.float32)

        def zb(i, carry):
            zero_v[i // (F // 16), pl.ds((i % (F // 16)) * 16, 16)] = z16
            return carry

        lax.fori_loop(0, 16 * (F // 16), zb, 0)

        def zcopy(q, carry):
            pltpu.sync_copy(zero_v, slab.at[pl.ds(s * RPB + q * 16, 16)])
            return carry

        lax.fori_loop(0, RPB // 16, zcopy, 0)

        @pl.when(s == 0)
        def _zero_tail():
            pltpu.sync_copy(zero_v.at[pl.ds(0, N - NS * RPB)],
                            slab.at[pl.ds(NS * RPB, N - NS * RPB)])

        plsc.subcore_barrier()

        if feat_split:
            stride = NS
            base = s
        else:
            stride = NC * NS
            base = s * NC + c
        nloop = (NCH + stride - 1) // stride

        def body(kk, carry):
            js = [(kk * R + b) * stride + base for b in range(R)]

            for b in range(R):
                @pl.when(js[b] < NCH)
                def _(b=b):
                    pltpu.async_copy(src_hbm.at[js[b]], src_v.at[b], isem)
                    pltpu.async_copy(dst_hbm.at[js[b]], dst_v.at[b], isem)

            for b in range(R):
                @pl.when(js[b] < NCH)
                def _(b=b):
                    pltpu.make_async_copy(src_hbm.at[js[b]], src_v.at[b],
                                          isem).wait()
                    pltpu.make_async_copy(dst_hbm.at[js[b]], dst_v.at[b],
                                          isem).wait()

            for b in range(R):
                @pl.when(js[b] < NCH)
                def _(b=b):
                    if feat_split:
                        off = c * N

                        def addoff(i, c2):
                            sl = pl.ds(i * 16, 16)
                            src_v[b, sl] = src_v[b, sl] + off
                            return c2

                        lax.fori_loop(0, CH // 16, addoff, 0)
                    pltpu.async_copy(g_hbm.at[src_v.at[b]], rows_v.at[b],
                                     gsem.at[b])

            for b in range(R):
                @pl.when(js[b] < NCH)
                def _(b=b):
                    pltpu.make_async_copy(g_hbm.at[src_v.at[b]],
                                          rows_v.at[b], gsem.at[b]).wait()
                    pltpu.async_copy(rows_v.at[b], slab.at[dst_v.at[b]],
                                     ssem, add=True)

            for b in range(R):
                @pl.when(js[b] < NCH)
                def _(b=b):
                    pltpu.make_async_copy(rows_v.at[b],
                                          slab.at[dst_v.at[b]], ssem).wait()

            return carry

        lax.fori_loop(0, (nloop + R - 1) // R, body, 0)
        plsc.subcore_barrier()
        pltpu.sync_copy(slab.at[pl.ds(s * RPB, RPB)],
                        out_hbm.at[c, pl.ds(s * RPB, RPB)])

        @pl.when(s == 0)
        def _drain_tail():
            pltpu.sync_copy(slab.at[pl.ds(NS * RPB, N - NS * RPB)],
                            out_hbm.at[c, pl.ds(NS * RPB, N - NS * RPB)])

    return k


_PROP_CACHE = {}


def _prop_feat(src2d, dst2d, g):
    if True not in _PROP_CACHE:
        _PROP_CACHE[True] = _make_prop(True)
    return _PROP_CACHE[True](src2d, dst2d, g)


def _prop_edge(src2d, dst2d, g):
    if False not in _PROP_CACHE:
        _PROP_CACHE[False] = _make_prop(False)
    return _PROP_CACHE[False](src2d, dst2d, g)


_BR = 2000


def _dinv_call(partials):
    def body(p_ref, o_ref):
        deg = jnp.sum(p_ref[...], axis=0) + 1.0
        o_ref[...] = lax.rsqrt(deg)[:, None]

    return pl.pallas_call(
        body,
        out_shape=jax.ShapeDtypeStruct((N, 1), jnp.float32),
    )(partials)


def _lin1_call(x, W1, dinv):
    def body(x_ref, w_ref, d_ref, o_ref):
        o_ref[...] = (d_ref[...] * jnp.dot(
            x_ref[...], w_ref[...], preferred_element_type=jnp.float32))[None]

    return pl.pallas_call(
        body,
        grid=(NC, N // _BR),
        in_specs=[
            pl.BlockSpec((_BR, 128), lambda c, i: (i, 0)),
            pl.BlockSpec((128, F), lambda c, i: (0, c)),
            pl.BlockSpec((_BR, 1), lambda c, i: (i, 0)),
        ],
        out_specs=pl.BlockSpec((1, _BR, F), lambda c, i: (c, i, 0)),
        out_shape=jax.ShapeDtypeStruct((NC, N, F), jnp.float32),
    )(x, W1, dinv)


def _lin2_call(A1, g1, dinv, b1r, W2r):
    def body(a_ref, g_ref, d_ref, b_ref, w_ref, o_ref):
        d = d_ref[...]
        u0 = jnp.maximum(d * (a_ref[0] + g_ref[0]) + b_ref[0], 0.0)
        u1 = jnp.maximum(d * (a_ref[1] + g_ref[1]) + b_ref[1], 0.0)
        acc = jnp.dot(u0, w_ref[0], preferred_element_type=jnp.float32)
        acc = acc + jnp.dot(u1, w_ref[1], preferred_element_type=jnp.float32)
        o_ref[...] = d * acc

    return pl.pallas_call(
        body,
        grid=(N // _BR,),
        in_specs=[
            pl.BlockSpec((NC, _BR, F), lambda i: (0, i, 0)),
            pl.BlockSpec((NC, _BR, F), lambda i: (0, i, 0)),
            pl.BlockSpec((_BR, 1), lambda i: (i, 0)),
            pl.BlockSpec((NC, 1, F), lambda i: (0, 0, 0)),
            pl.BlockSpec((NC, F, F), lambda i: (0, 0, 0)),
        ],
        out_specs=pl.BlockSpec((_BR, F), lambda i: (i, 0)),
        out_shape=jax.ShapeDtypeStruct((N, F), jnp.float32),
    )(A1, g1, dinv, b1r, W2r)


def _lin_mid_call(S, g, dinv, br, W):
    def body(s_ref, g_ref, d_ref, b_ref, w_ref, o_ref):
        d = d_ref[...]
        u = jnp.maximum(d * (s_ref[0] + s_ref[1] + g_ref[...]) + b_ref[...],
                        0.0)
        o_ref[...] = d * jnp.dot(u, w_ref[...],
                                 preferred_element_type=jnp.float32)

    return pl.pallas_call(
        body,
        grid=(N // _BR,),
        in_specs=[
            pl.BlockSpec((NC, _BR, F), lambda i: (0, i, 0)),
            pl.BlockSpec((_BR, F), lambda i: (i, 0)),
            pl.BlockSpec((_BR, 1), lambda i: (i, 0)),
            pl.BlockSpec((1, F), lambda i: (0, 0)),
            pl.BlockSpec((F, F), lambda i: (0, 0)),
        ],
        out_specs=pl.BlockSpec((_BR, F), lambda i: (i, 0)),
        out_shape=jax.ShapeDtypeStruct((N, F), jnp.float32),
    )(S, g, dinv, br, W)


def _head_call(S, g, dinv, b5r, Wm1p, bm1p, Wm2p, bm2p):
    def body(s_ref, g_ref, d_ref, b_ref, w1_ref, c1_ref, w2_ref, c2_ref,
             o_ref):
        d = d_ref[...]
        u = jnp.maximum(d * (s_ref[0] + s_ref[1] + g_ref[...]) + b_ref[...],
                        0.0)
        h2 = jnp.maximum(
            jnp.dot(u, w1_ref[...], preferred_element_type=jnp.float32)
            + c1_ref[...], 0.0)
        o_ref[...] = jnp.dot(
            h2, w2_ref[...], preferred_element_type=jnp.float32) + c2_ref[...]

    return pl.pallas_call(
        body,
        grid=(N // _BR,),
        in_specs=[
            pl.BlockSpec((NC, _BR, F), lambda i: (0, i, 0)),
            pl.BlockSpec((_BR, F), lambda i: (i, 0)),
            pl.BlockSpec((_BR, 1), lambda i: (i, 0)),
            pl.BlockSpec((1, F), lambda i: (0, 0)),
            pl.BlockSpec((F, F), lambda i: (0, 0)),
            pl.BlockSpec((1, F), lambda i: (0, 0)),
            pl.BlockSpec((F, F), lambda i: (0, 0)),
            pl.BlockSpec((1, F), lambda i: (0, 0)),
        ],
        out_specs=pl.BlockSpec((_BR, F), lambda i: (i, 0)),
        out_shape=jax.ShapeDtypeStruct((N, F), jnp.float32),
    )(S, g, dinv, b5r, Wm1p, bm1p, Wm2p, bm2p)


def kernel(x, edge_index, batch, W1, b1, W2, b2, W3, b3, W4, b4, W5, b5,
           Wm1, bm1, Wm2, bm2):
    del batch
    src2d = edge_index[0].reshape(NCH, CH)
    dst2d = edge_index[1].reshape(NCH, CH)

    partials = _degree_call(dst2d)
    dinv = _dinv_call(partials)

    g1 = _lin1_call(x, W1, dinv)
    A1 = _prop_feat(src2d, dst2d, g1.reshape(NC * N, F))
    g2 = _lin2_call(A1, g1, dinv, b1.reshape(NC, 1, F),
                    W2.reshape(NC, F, F))

    g = g2
    for (bl, Wn) in ((b2, W3), (b3, W4), (b4, W5)):
        S = _prop_edge(src2d, dst2d, g)
        g = _lin_mid_call(S, g, dinv, bl.reshape(1, F), Wn)

    S5 = _prop_edge(src2d, dst2d, g)

    hm = Wm1.shape[1]
    Wm1p = jnp.pad(Wm1, ((0, 0), (0, F - hm)))
    bm1p = jnp.pad(bm1, (0, F - hm)).reshape(1, F)
    Wm2p = jnp.pad(Wm2, ((0, F - hm), (0, F - Wm2.shape[1])))
    bm2p = jnp.pad(bm2, (0, F - Wm2.shape[1])).reshape(1, F)

    out = _head_call(S5, g, dinv, b5.reshape(1, F), Wm1p, bm1p, Wm2p, bm2p)
    return out[:, :Wm2.shape[1]]


# speedup vs baseline: 15.8421x; 1.0104x over previous
import functools

import jax
import jax.numpy as jnp
from jax import lax
from jax.experimental import pallas as pl
from jax.experimental.pallas import tpu as pltpu
from jax.experimental.pallas import tpu_sc as plsc

N = 10000
E = 320000
NC = 2
NS = 16
CH = 128
NCH = E // CH
RPB = 624
F = 128


def _mesh():
    return plsc.VectorSubcoreMesh(core_axis_name="c", subcore_axis_name="s")


def _degree_call(dst2d):
    @functools.partial(
        pl.kernel,
        mesh=_mesh(),
        out_type=jax.ShapeDtypeStruct((NC * NS, N), jnp.float32),
        compiler_params=pltpu.CompilerParams(needs_layout_passes=False),
        scratch_types=[
            pltpu.VMEM((N,), jnp.float32),
            pltpu.VMEM((CH,), jnp.int32),
        ],
    )
    def k(dst_hbm, out_hbm, hist_v, idx_v):
        c = lax.axis_index("c")
        s = lax.axis_index("s")
        tid = s * NC + c
        z16 = jnp.zeros((16,), jnp.float32)

        def zero_body(i, carry):
            hist_v[pl.ds(i * 16, 16)] = z16
            return carry

        lax.fori_loop(0, N // 16, zero_body, 0)

        ones = jnp.ones((16,), jnp.float32)
        nloop = (NCH + NC * NS - 1) // (NC * NS)

        def chunk_body(kk, carry):
            j = kk * (NC * NS) + tid

            @pl.when(j < NCH)
            def _():
                pltpu.sync_copy(dst_hbm.at[j], idx_v)

                def inner(i, c2):
                    idx = idx_v[pl.ds(i * 16, 16)]
                    plsc.addupdate_scatter(hist_v, [idx], ones)
                    return c2

                lax.fori_loop(0, CH // 16, inner, 0)

            return carry

        lax.fori_loop(0, nloop, chunk_body, 0)
        pltpu.sync_copy(hist_v, out_hbm.at[tid])

    return k(dst2d)


def _make_prop(feat_split):
    R = 2

    @functools.partial(
        pl.kernel,
        mesh=_mesh(),
        out_type=jax.ShapeDtypeStruct((NC, N, F), jnp.float32),
        scratch_types=[
            pltpu.VMEM((R, CH), jnp.int32),
            pltpu.VMEM((R, CH), jnp.int32),
            pltpu.VMEM((R, CH, F), jnp.float32),
            pltpu.VMEM((16, F), jnp.float32),
            pltpu.VMEM_SHARED((N, F), jnp.float32),
            pltpu.SemaphoreType.DMA((R,)),
            pltpu.SemaphoreType.DMA((R,)),
            pltpu.SemaphoreType.DMA,
        ],
    )
    def k(src_hbm, dst_hbm, g_hbm, out_hbm, src_v, dst_v, rows_v, zero_v,
          slab, gsem, ssem, isem):
        c = lax.axis_index("c")
        s = lax.axis_index("s")
        z16 = jnp.zeros((16,), jnp.float32)

        def zb(i, carry):
            zero_v[i // (F // 16), pl.ds((i % (F // 16)) * 16, 16)] = z16
            return carry

        lax.fori_loop(0, 16 * (F // 16), zb, 0)

        def zcopy(q, carry):
            pltpu.sync_copy(zero_v, slab.at[pl.ds(s * RPB + q * 16, 16)])
            return carry

        lax.fori_loop(0, RPB // 16, zcopy, 0)

        @pl.when(s == 0)
        def _zero_tail():
            pltpu.sync_copy(zero_v.at[pl.ds(0, N - NS * RPB)],
                            slab.at[pl.ds(NS * RPB, N - NS * RPB)])

        plsc.subcore_barrier()

        if feat_split:
            stride = NS
            base = s
        else:
            stride = NC * NS
            base = s * NC + c
        nloop = (NCH + stride - 1) // stride

        def body(kk, carry):
            js = [(kk * R + b) * stride + base for b in range(R)]

            for b in range(R):
                @pl.when(js[b] - R * stride >= 0)
                def _(b=b):
                    pltpu.make_async_copy(rows_v.at[b],
                                          slab.at[dst_v.at[b]],
                                          ssem.at[b]).wait()

            for b in range(R):
                @pl.when(js[b] < NCH)
                def _(b=b):
                    pltpu.async_copy(src_hbm.at[js[b]], src_v.at[b], isem)
                    pltpu.async_copy(dst_hbm.at[js[b]], dst_v.at[b], isem)

            for b in range(R):
                @pl.when(js[b] < NCH)
                def _(b=b):
                    pltpu.make_async_copy(src_hbm.at[js[b]], src_v.at[b],
                                          isem).wait()
                    pltpu.make_async_copy(dst_hbm.at[js[b]], dst_v.at[b],
                                          isem).wait()
                    if feat_split:
                        off = c * N

                        def addoff(i, c2):
                            sl = pl.ds(i * 16, 16)
                            src_v[b, sl] = src_v[b, sl] + off
                            return c2

                        lax.fori_loop(0, CH // 16, addoff, 0)
                    pltpu.async_copy(g_hbm.at[src_v.at[b]], rows_v.at[b],
                                     gsem.at[b])

            for b in range(R):
                @pl.when(js[b] < NCH)
                def _(b=b):
                    pltpu.make_async_copy(g_hbm.at[src_v.at[b]],
                                          rows_v.at[b], gsem.at[b]).wait()
                    pltpu.async_copy(rows_v.at[b], slab.at[dst_v.at[b]],
                                     ssem.at[b], add=True)

            return carry

        nbody = (nloop + R - 1) // R
        lax.fori_loop(0, nbody, body, 0)
        for b in range(R):
            @pl.when((nbody - 1) * R * stride + b * stride + base < NCH)
            def _(b=b):
                pltpu.make_async_copy(rows_v.at[b], slab.at[dst_v.at[b]],
                                      ssem.at[b]).wait()
        plsc.subcore_barrier()
        pltpu.sync_copy(slab.at[pl.ds(s * RPB, RPB)],
                        out_hbm.at[c, pl.ds(s * RPB, RPB)])

        @pl.when(s == 0)
        def _drain_tail():
            pltpu.sync_copy(slab.at[pl.ds(NS * RPB, N - NS * RPB)],
                            out_hbm.at[c, pl.ds(NS * RPB, N - NS * RPB)])

    return k


_PROP_CACHE = {}


def _prop_feat(src2d, dst2d, g):
    if True not in _PROP_CACHE:
        _PROP_CACHE[True] = _make_prop(True)
    return _PROP_CACHE[True](src2d, dst2d, g)


def _prop_edge(src2d, dst2d, g):
    if False not in _PROP_CACHE:
        _PROP_CACHE[False] = _make_prop(False)
    return _PROP_CACHE[False](src2d, dst2d, g)


_BR = 2000


def _dinv_call(partials):
    def body(p_ref, o_ref):
        deg = jnp.sum(p_ref[...], axis=0) + 1.0
        o_ref[...] = lax.rsqrt(deg)[:, None]

    return pl.pallas_call(
        body,
        out_shape=jax.ShapeDtypeStruct((N, 1), jnp.float32),
    )(partials)


def _lin1_call(x, W1, dinv):
    def body(x_ref, w_ref, d_ref, o_ref):
        o_ref[...] = (d_ref[...] * jnp.dot(
            x_ref[...], w_ref[...], preferred_element_type=jnp.float32))[None]

    return pl.pallas_call(
        body,
        grid=(NC, N // _BR),
        in_specs=[
            pl.BlockSpec((_BR, 128), lambda c, i: (i, 0)),
            pl.BlockSpec((128, F), lambda c, i: (0, c)),
            pl.BlockSpec((_BR, 1), lambda c, i: (i, 0)),
        ],
        out_specs=pl.BlockSpec((1, _BR, F), lambda c, i: (c, i, 0)),
        out_shape=jax.ShapeDtypeStruct((NC, N, F), jnp.float32),
    )(x, W1, dinv)


def _lin2_call(A1, g1, dinv, b1r, W2r):
    def body(a_ref, g_ref, d_ref, b_ref, w_ref, o_ref):
        d = d_ref[...]
        u0 = jnp.maximum(d * (a_ref[0] + g_ref[0]) + b_ref[0], 0.0)
        u1 = jnp.maximum(d * (a_ref[1] + g_ref[1]) + b_ref[1], 0.0)
        acc = jnp.dot(u0, w_ref[0], preferred_element_type=jnp.float32)
        acc = acc + jnp.dot(u1, w_ref[1], preferred_element_type=jnp.float32)
        o_ref[...] = d * acc

    return pl.pallas_call(
        body,
        grid=(N // _BR,),
        in_specs=[
            pl.BlockSpec((NC, _BR, F), lambda i: (0, i, 0)),
            pl.BlockSpec((NC, _BR, F), lambda i: (0, i, 0)),
            pl.BlockSpec((_BR, 1), lambda i: (i, 0)),
            pl.BlockSpec((NC, 1, F), lambda i: (0, 0, 0)),
            pl.BlockSpec((NC, F, F), lambda i: (0, 0, 0)),
        ],
        out_specs=pl.BlockSpec((_BR, F), lambda i: (i, 0)),
        out_shape=jax.ShapeDtypeStruct((N, F), jnp.float32),
    )(A1, g1, dinv, b1r, W2r)


def _lin_mid_call(S, g, dinv, br, W):
    def body(s_ref, g_ref, d_ref, b_ref, w_ref, o_ref):
        d = d_ref[...]
        u = jnp.maximum(d * (s_ref[0] + s_ref[1] + g_ref[...]) + b_ref[...],
                        0.0)
        o_ref[...] = d * jnp.dot(u, w_ref[...],
                                 preferred_element_type=jnp.float32)

    return pl.pallas_call(
        body,
        grid=(N // _BR,),
        in_specs=[
            pl.BlockSpec((NC, _BR, F), lambda i: (0, i, 0)),
            pl.BlockSpec((_BR, F), lambda i: (i, 0)),
            pl.BlockSpec((_BR, 1), lambda i: (i, 0)),
            pl.BlockSpec((1, F), lambda i: (0, 0)),
            pl.BlockSpec((F, F), lambda i: (0, 0)),
        ],
        out_specs=pl.BlockSpec((_BR, F), lambda i: (i, 0)),
        out_shape=jax.ShapeDtypeStruct((N, F), jnp.float32),
    )(S, g, dinv, br, W)


def _head_call(S, g, dinv, b5r, Wm1p, bm1p, Wm2p, bm2p):
    def body(s_ref, g_ref, d_ref, b_ref, w1_ref, c1_ref, w2_ref, c2_ref,
             o_ref):
        d = d_ref[...]
        u = jnp.maximum(d * (s_ref[0] + s_ref[1] + g_ref[...]) + b_ref[...],
                        0.0)
        h2 = jnp.maximum(
            jnp.dot(u, w1_ref[...], preferred_element_type=jnp.float32)
            + c1_ref[...], 0.0)
        o_ref[...] = jnp.dot(
            h2, w2_ref[...], preferred_element_type=jnp.float32) + c2_ref[...]

    return pl.pallas_call(
        body,
        grid=(N // _BR,),
        in_specs=[
            pl.BlockSpec((NC, _BR, F), lambda i: (0, i, 0)),
            pl.BlockSpec((_BR, F), lambda i: (i, 0)),
            pl.BlockSpec((_BR, 1), lambda i: (i, 0)),
            pl.BlockSpec((1, F), lambda i: (0, 0)),
            pl.BlockSpec((F, F), lambda i: (0, 0)),
            pl.BlockSpec((1, F), lambda i: (0, 0)),
            pl.BlockSpec((F, F), lambda i: (0, 0)),
            pl.BlockSpec((1, F), lambda i: (0, 0)),
        ],
        out_specs=pl.BlockSpec((_BR, F), lambda i: (i, 0)),
        out_shape=jax.ShapeDtypeStruct((N, F), jnp.float32),
    )(S, g, dinv, b5r, Wm1p, bm1p, Wm2p, bm2p)


def kernel(x, edge_index, batch, W1, b1, W2, b2, W3, b3, W4, b4, W5, b5,
           Wm1, bm1, Wm2, bm2):
    del batch
    src2d = edge_index[0].reshape(NCH, CH)
    dst2d = edge_index[1].reshape(NCH, CH)

    partials = _degree_call(dst2d)
    dinv = _dinv_call(partials)

    g1 = _lin1_call(x, W1, dinv)
    A1 = _prop_feat(src2d, dst2d, g1.reshape(NC * N, F))
    g2 = _lin2_call(A1, g1, dinv, b1.reshape(NC, 1, F),
                    W2.reshape(NC, F, F))

    g = g2
    for (bl, Wn) in ((b2, W3), (b3, W4), (b4, W5)):
        S = _prop_edge(src2d, dst2d, g)
        g = _lin_mid_call(S, g, dinv, bl.reshape(1, F), Wn)

    S5 = _prop_edge(src2d, dst2d, g)

    hm = Wm1.shape[1]
    Wm1p = jnp.pad(Wm1, ((0, 0), (0, F - hm)))
    bm1p = jnp.pad(bm1, (0, F - hm)).reshape(1, F)
    Wm2p = jnp.pad(Wm2, ((0, F - hm), (0, F - Wm2.shape[1])))
    bm2p = jnp.pad(bm2, (0, F - Wm2.shape[1])).reshape(1, F)

    out = _head_call(S5, g, dinv, b5.reshape(1, F), Wm1p, bm1p, Wm2p, bm2p)
    return out[:, :Wm2.shape[1]]


# speedup vs baseline: 16.7376x; 1.0565x over previous
import functools

import jax
import jax.numpy as jnp
from jax import lax
from jax.experimental import pallas as pl
from jax.experimental.pallas import tpu as pltpu
from jax.experimental.pallas import tpu_sc as plsc

N = 10000
E = 320000
NC = 2
NS = 16
CH = 128
NCH = E // CH
RPB = 624
F = 128


def _mesh():
    return plsc.VectorSubcoreMesh(core_axis_name="c", subcore_axis_name="s")


def _degree_call(dst2d):
    @functools.partial(
        pl.kernel,
        mesh=_mesh(),
        out_type=jax.ShapeDtypeStruct((NC * NS, N), jnp.float32),
        compiler_params=pltpu.CompilerParams(needs_layout_passes=False),
        scratch_types=[
            pltpu.VMEM((N,), jnp.float32),
            pltpu.VMEM((CH,), jnp.int32),
        ],
    )
    def k(dst_hbm, out_hbm, hist_v, idx_v):
        c = lax.axis_index("c")
        s = lax.axis_index("s")
        tid = s * NC + c
        z16 = jnp.zeros((16,), jnp.float32)

        def zero_body(i, carry):
            hist_v[pl.ds(i * 16, 16)] = z16
            return carry

        lax.fori_loop(0, N // 16, zero_body, 0)

        ones = jnp.ones((16,), jnp.float32)
        nloop = (NCH + NC * NS - 1) // (NC * NS)

        def chunk_body(kk, carry):
            j = kk * (NC * NS) + tid

            @pl.when(j < NCH)
            def _():
                pltpu.sync_copy(dst_hbm.at[j], idx_v)

                def inner(i, c2):
                    idx = idx_v[pl.ds(i * 16, 16)]
                    plsc.addupdate_scatter(hist_v, [idx], ones)
                    return c2

                lax.fori_loop(0, CH // 16, inner, 0)

            return carry

        lax.fori_loop(0, nloop, chunk_body, 0)
        pltpu.sync_copy(hist_v, out_hbm.at[tid])

    return k(dst2d)


def _make_prop(feat_split):
    R = 3

    @functools.partial(
        pl.kernel,
        mesh=_mesh(),
        out_type=jax.ShapeDtypeStruct((NC, N, F), jnp.float32),
        scratch_types=[
            pltpu.VMEM((R, CH), jnp.int32),
            pltpu.VMEM((R, CH), jnp.int32),
            pltpu.VMEM((R, CH, F), jnp.float32),
            pltpu.VMEM_SHARED((N, F), jnp.float32),
            pltpu.SemaphoreType.DMA((R,)),
            pltpu.SemaphoreType.DMA((R,)),
            pltpu.SemaphoreType.DMA,
        ],
    )
    def k(src_hbm, dst_hbm, g_hbm, out_hbm, src_v, dst_v, rows_v,
          slab, gsem, ssem, isem):
        c = lax.axis_index("c")
        s = lax.axis_index("s")
        z16 = jnp.zeros((16,), jnp.float32)

        def zb(i, carry):
            rows_v[0, i // (F // 16), pl.ds((i % (F // 16)) * 16, 16)] = z16
            return carry

        lax.fori_loop(0, 8 * (F // 16), zb, 0)
        zrows = rows_v.at[0, pl.ds(0, 8)]

        def zcopy(q, carry):
            pltpu.sync_copy(zrows, slab.at[pl.ds(s * RPB + q * 8, 8)])
            return carry

        lax.fori_loop(0, RPB // 8, zcopy, 0)

        @pl.when(s == 0)
        def _zero_tail():
            pltpu.sync_copy(zrows, slab.at[pl.ds(NS * RPB, 8)])
            pltpu.sync_copy(zrows, slab.at[pl.ds(NS * RPB + 8, 8)])

        plsc.subcore_barrier()

        if feat_split:
            stride = NS
            base = s
        else:
            stride = NC * NS
            base = s * NC + c
        nloop = (NCH + stride - 1) // stride

        def body(kk, carry):
            js = [(kk * R + b) * stride + base for b in range(R)]

            for b in range(R):
                @pl.when(js[b] - R * stride >= 0)
                def _(b=b):
                    pltpu.make_async_copy(rows_v.at[b],
                                          slab.at[dst_v.at[b]],
                                          ssem.at[b]).wait()

            for b in range(R):
                @pl.when(js[b] < NCH)
                def _(b=b):
                    pltpu.async_copy(src_hbm.at[js[b]], src_v.at[b], isem)
                    pltpu.async_copy(dst_hbm.at[js[b]], dst_v.at[b], isem)

            for b in range(R):
                @pl.when(js[b] < NCH)
                def _(b=b):
                    pltpu.make_async_copy(src_hbm.at[js[b]], src_v.at[b],
                                          isem).wait()
                    pltpu.make_async_copy(dst_hbm.at[js[b]], dst_v.at[b],
                                          isem).wait()
                    if feat_split:
                        off = c * N

                        def addoff(i, c2):
                            sl = pl.ds(i * 16, 16)
                            src_v[b, sl] = src_v[b, sl] + off
                            return c2

                        lax.fori_loop(0, CH // 16, addoff, 0)
                    pltpu.async_copy(g_hbm.at[src_v.at[b]], rows_v.at[b],
                                     gsem.at[b])

            for b in range(R):
                @pl.when(js[b] < NCH)
                def _(b=b):
                    pltpu.make_async_copy(g_hbm.at[src_v.at[b]],
                                          rows_v.at[b], gsem.at[b]).wait()
                    pltpu.async_copy(rows_v.at[b], slab.at[dst_v.at[b]],
                                     ssem.at[b], add=True)

            return carry

        nbody = (nloop + R - 1) // R
        lax.fori_loop(0, nbody, body, 0)
        for b in range(R):
            @pl.when((nbody - 1) * R * stride + b * stride + base < NCH)
            def _(b=b):
                pltpu.make_async_copy(rows_v.at[b], slab.at[dst_v.at[b]],
                                      ssem.at[b]).wait()
        plsc.subcore_barrier()
        pltpu.sync_copy(slab.at[pl.ds(s * RPB, RPB)],
                        out_hbm.at[c, pl.ds(s * RPB, RPB)])

        @pl.when(s == 0)
        def _drain_tail():
            pltpu.sync_copy(slab.at[pl.ds(NS * RPB, N - NS * RPB)],
                            out_hbm.at[c, pl.ds(NS * RPB, N - NS * RPB)])

    return k


_PROP_CACHE = {}


def _prop_feat(src2d, dst2d, g):
    if True not in _PROP_CACHE:
        _PROP_CACHE[True] = _make_prop(True)
    return _PROP_CACHE[True](src2d, dst2d, g)


def _prop_edge(src2d, dst2d, g):
    if False not in _PROP_CACHE:
        _PROP_CACHE[False] = _make_prop(False)
    return _PROP_CACHE[False](src2d, dst2d, g)


_BR = 2000


def _dinv_call(partials):
    def body(p_ref, o_ref):
        deg = jnp.sum(p_ref[...], axis=0) + 1.0
        o_ref[...] = lax.rsqrt(deg)[:, None]

    return pl.pallas_call(
        body,
        out_shape=jax.ShapeDtypeStruct((N, 1), jnp.float32),
    )(partials)


def _lin1_call(x, W1, dinv):
    def body(x_ref, w_ref, d_ref, o_ref):
        o_ref[...] = (d_ref[...] * jnp.dot(
            x_ref[...], w_ref[...], preferred_element_type=jnp.float32))[None]

    return pl.pallas_call(
        body,
        grid=(NC, N // _BR),
        in_specs=[
            pl.BlockSpec((_BR, 128), lambda c, i: (i, 0)),
            pl.BlockSpec((128, F), lambda c, i: (0, c)),
            pl.BlockSpec((_BR, 1), lambda c, i: (i, 0)),
        ],
        out_specs=pl.BlockSpec((1, _BR, F), lambda c, i: (c, i, 0)),
        out_shape=jax.ShapeDtypeStruct((NC, N, F), jnp.float32),
    )(x, W1, dinv)


def _lin2_call(A1, g1, dinv, b1r, W2r):
    def body(a_ref, g_ref, d_ref, b_ref, w_ref, o_ref):
        d = d_ref[...]
        u0 = jnp.maximum(d * (a_ref[0] + g_ref[0]) + b_ref[0], 0.0)
        u1 = jnp.maximum(d * (a_ref[1] + g_ref[1]) + b_ref[1], 0.0)
        acc = jnp.dot(u0, w_ref[0], preferred_element_type=jnp.float32)
        acc = acc + jnp.dot(u1, w_ref[1], preferred_element_type=jnp.float32)
        o_ref[...] = d * acc

    return pl.pallas_call(
        body,
        grid=(N // _BR,),
        in_specs=[
            pl.BlockSpec((NC, _BR, F), lambda i: (0, i, 0)),
            pl.BlockSpec((NC, _BR, F), lambda i: (0, i, 0)),
            pl.BlockSpec((_BR, 1), lambda i: (i, 0)),
            pl.BlockSpec((NC, 1, F), lambda i: (0, 0, 0)),
            pl.BlockSpec((NC, F, F), lambda i: (0, 0, 0)),
        ],
        out_specs=pl.BlockSpec((_BR, F), lambda i: (i, 0)),
        out_shape=jax.ShapeDtypeStruct((N, F), jnp.float32),
    )(A1, g1, dinv, b1r, W2r)


def _lin_mid_call(S, g, dinv, br, W):
    def body(s_ref, g_ref, d_ref, b_ref, w_ref, o_ref):
        d = d_ref[...]
        u = jnp.maximum(d * (s_ref[0] + s_ref[1] + g_ref[...]) + b_ref[...],
                        0.0)
        o_ref[...] = d * jnp.dot(u, w_ref[...],
                                 preferred_element_type=jnp.float32)

    return pl.pallas_call(
        body,
        grid=(N // _BR,),
        in_specs=[
            pl.BlockSpec((NC, _BR, F), lambda i: (0, i, 0)),
            pl.BlockSpec((_BR, F), lambda i: (i, 0)),
            pl.BlockSpec((_BR, 1), lambda i: (i, 0)),
            pl.BlockSpec((1, F), lambda i: (0, 0)),
            pl.BlockSpec((F, F), lambda i: (0, 0)),
        ],
        out_specs=pl.BlockSpec((_BR, F), lambda i: (i, 0)),
        out_shape=jax.ShapeDtypeStruct((N, F), jnp.float32),
    )(S, g, dinv, br, W)


def _head_call(S, g, dinv, b5r, Wm1p, bm1p, Wm2p, bm2p):
    def body(s_ref, g_ref, d_ref, b_ref, w1_ref, c1_ref, w2_ref, c2_ref,
             o_ref):
        d = d_ref[...]
        u = jnp.maximum(d * (s_ref[0] + s_ref[1] + g_ref[...]) + b_ref[...],
                        0.0)
        h2 = jnp.maximum(
            jnp.dot(u, w1_ref[...], preferred_element_type=jnp.float32)
            + c1_ref[...], 0.0)
        o_ref[...] = jnp.dot(
            h2, w2_ref[...], preferred_element_type=jnp.float32) + c2_ref[...]

    return pl.pallas_call(
        body,
        grid=(N // _BR,),
        in_specs=[
            pl.BlockSpec((NC, _BR, F), lambda i: (0, i, 0)),
            pl.BlockSpec((_BR, F), lambda i: (i, 0)),
            pl.BlockSpec((_BR, 1), lambda i: (i, 0)),
            pl.BlockSpec((1, F), lambda i: (0, 0)),
            pl.BlockSpec((F, F), lambda i: (0, 0)),
            pl.BlockSpec((1, F), lambda i: (0, 0)),
            pl.BlockSpec((F, F), lambda i: (0, 0)),
            pl.BlockSpec((1, F), lambda i: (0, 0)),
        ],
        out_specs=pl.BlockSpec((_BR, F), lambda i: (i, 0)),
        out_shape=jax.ShapeDtypeStruct((N, F), jnp.float32),
    )(S, g, dinv, b5r, Wm1p, bm1p, Wm2p, bm2p)


def kernel(x, edge_index, batch, W1, b1, W2, b2, W3, b3, W4, b4, W5, b5,
           Wm1, bm1, Wm2, bm2):
    del batch
    src2d = edge_index[0].reshape(NCH, CH)
    dst2d = edge_index[1].reshape(NCH, CH)

    partials = _degree_call(dst2d)
    dinv = _dinv_call(partials)

    g1 = _lin1_call(x, W1, dinv)
    A1 = _prop_feat(src2d, dst2d, g1.reshape(NC * N, F))
    g2 = _lin2_call(A1, g1, dinv, b1.reshape(NC, 1, F),
                    W2.reshape(NC, F, F))

    g = g2
    for (bl, Wn) in ((b2, W3), (b3, W4), (b4, W5)):
        S = _prop_edge(src2d, dst2d, g)
        g = _lin_mid_call(S, g, dinv, bl.reshape(1, F), Wn)

    S5 = _prop_edge(src2d, dst2d, g)

    hm = Wm1.shape[1]
    Wm1p = jnp.pad(Wm1, ((0, 0), (0, F - hm)))
    bm1p = jnp.pad(bm1, (0, F - hm)).reshape(1, F)
    Wm2p = jnp.pad(Wm2, ((0, F - hm), (0, F - Wm2.shape[1])))
    bm2p = jnp.pad(bm2, (0, F - Wm2.shape[1])).reshape(1, F)

    out = _head_call(S5, g, dinv, b5.reshape(1, F), Wm1p, bm1p, Wm2p, bm2p)
    return out[:, :Wm2.shape[1]]


# speedup vs baseline: 19.4275x; 1.1607x over previous
import functools

import jax
import jax.numpy as jnp
from jax import lax
from jax.experimental import pallas as pl
from jax.experimental.pallas import tpu as pltpu
from jax.experimental.pallas import tpu_sc as plsc

N = 10000
E = 320000
NC = 2
NS = 16
CH = 128
NCH = E // CH
RPB = 624
F = 128


def _mesh():
    return plsc.VectorSubcoreMesh(core_axis_name="c", subcore_axis_name="s")


def _degree_call(dst2d):
    @functools.partial(
        pl.kernel,
        mesh=_mesh(),
        out_type=jax.ShapeDtypeStruct((NC * NS, N), jnp.float32),
        compiler_params=pltpu.CompilerParams(needs_layout_passes=False),
        scratch_types=[
            pltpu.VMEM((N,), jnp.float32),
            pltpu.VMEM((CH,), jnp.int32),
        ],
    )
    def k(dst_hbm, out_hbm, hist_v, idx_v):
        c = lax.axis_index("c")
        s = lax.axis_index("s")
        tid = s * NC + c
        z16 = jnp.zeros((16,), jnp.float32)

        def zero_body(i, carry):
            hist_v[pl.ds(i * 16, 16)] = z16
            return carry

        lax.fori_loop(0, N // 16, zero_body, 0)

        ones = jnp.ones((16,), jnp.float32)
        nloop = (NCH + NC * NS - 1) // (NC * NS)

        def chunk_body(kk, carry):
            j = kk * (NC * NS) + tid

            @pl.when(j < NCH)
            def _():
                pltpu.sync_copy(dst_hbm.at[j], idx_v)

                def inner(i, c2):
                    idx = idx_v[pl.ds(i * 16, 16)]
                    plsc.addupdate_scatter(hist_v, [idx], ones)
                    return c2

                lax.fori_loop(0, CH // 16, inner, 0)

            return carry

        lax.fori_loop(0, nloop, chunk_body, 0)
        pltpu.sync_copy(hist_v, out_hbm.at[tid])

    return k(dst2d)


def _make_prop():
    R = 3

    @functools.partial(
        pl.kernel,
        mesh=_mesh(),
        out_type=jax.ShapeDtypeStruct((NC, N, F), jnp.float32),
        scratch_types=[
            pltpu.VMEM((R, CH), jnp.int32),
            pltpu.VMEM((R, CH), jnp.int32),
            pltpu.VMEM((R, CH, F), jnp.float32),
            pltpu.VMEM_SHARED((N, F), jnp.float32),
            pltpu.SemaphoreType.DMA((R,)),
            pltpu.SemaphoreType.DMA((R,)),
            pltpu.SemaphoreType.DMA,
        ],
    )
    def k(src_hbm, dst_hbm, g_hbm, out_hbm, src_v, dst_v, rows_v,
          slab, gsem, ssem, isem):
        c = lax.axis_index("c")
        s = lax.axis_index("s")
        z16 = jnp.zeros((16,), jnp.float32)

        def zb(i, carry):
            rows_v[0, i // (F // 16), pl.ds((i % (F // 16)) * 16, 16)] = z16
            return carry

        lax.fori_loop(0, 8 * (F // 16), zb, 0)
        zrows = rows_v.at[0, pl.ds(0, 8)]

        def zcopy(q, carry):
            pltpu.sync_copy(zrows, slab.at[pl.ds(s * RPB + q * 8, 8)])
            return carry

        lax.fori_loop(0, RPB // 8, zcopy, 0)

        @pl.when(s == 0)
        def _zero_tail():
            pltpu.sync_copy(zrows, slab.at[pl.ds(NS * RPB, 8)])
            pltpu.sync_copy(zrows, slab.at[pl.ds(NS * RPB + 8, 8)])

        plsc.subcore_barrier()

        stride = NC * NS
        base = s * NC + c
        nloop = (NCH + stride - 1) // stride

        def body(kk, carry):
            js = [(kk * R + b) * stride + base for b in range(R)]

            for b in range(R):
                @pl.when(js[b] - R * stride >= 0)
                def _(b=b):
                    pltpu.make_async_copy(rows_v.at[b],
                                          slab.at[dst_v.at[b]],
                                          ssem.at[b]).wait()

            for b in range(R):
                @pl.when(js[b] < NCH)
                def _(b=b):
                    pltpu.async_copy(src_hbm.at[js[b]], src_v.at[b], isem)
                    pltpu.async_copy(dst_hbm.at[js[b]], dst_v.at[b], isem)

            for b in range(R):
                @pl.when(js[b] < NCH)
                def _(b=b):
                    pltpu.make_async_copy(src_hbm.at[js[b]], src_v.at[b],
                                          isem).wait()
                    pltpu.make_async_copy(dst_hbm.at[js[b]], dst_v.at[b],
                                          isem).wait()
                    pltpu.async_copy(g_hbm.at[src_v.at[b]], rows_v.at[b],
                                     gsem.at[b])

            for b in range(R):
                @pl.when(js[b] < NCH)
                def _(b=b):
                    pltpu.make_async_copy(g_hbm.at[src_v.at[b]],
                                          rows_v.at[b], gsem.at[b]).wait()
                    pltpu.async_copy(rows_v.at[b], slab.at[dst_v.at[b]],
                                     ssem.at[b], add=True)

            return carry

        nbody = (nloop + R - 1) // R
        lax.fori_loop(0, nbody, body, 0)
        for b in range(R):
            @pl.when((nbody - 1) * R * stride + b * stride + base < NCH)
            def _(b=b):
                pltpu.make_async_copy(rows_v.at[b], slab.at[dst_v.at[b]],
                                      ssem.at[b]).wait()
        plsc.subcore_barrier()
        pltpu.sync_copy(slab.at[pl.ds(s * RPB, RPB)],
                        out_hbm.at[c, pl.ds(s * RPB, RPB)])

        @pl.when(s == 0)
        def _drain_tail():
            pltpu.sync_copy(slab.at[pl.ds(NS * RPB, N - NS * RPB)],
                            out_hbm.at[c, pl.ds(NS * RPB, N - NS * RPB)])

    return k


_PROP_CACHE = {}


def _prop_edge(src2d, dst2d, g):
    if 0 not in _PROP_CACHE:
        _PROP_CACHE[0] = _make_prop()
    return _PROP_CACHE[0](src2d, dst2d, g)


_BR = 2000


def _dinv_call(partials, x):
    def body(p_ref, x_ref, d_ref, xs_ref):
        deg = jnp.sum(p_ref[...], axis=0) + 1.0
        d = lax.rsqrt(deg)[:, None]
        d_ref[...] = d
        xs_ref[...] = d * x_ref[...]

    return pl.pallas_call(
        body,
        out_shape=(jax.ShapeDtypeStruct((N, 1), jnp.float32),
                   jax.ShapeDtypeStruct((N, 128), jnp.float32)),
    )(partials, x)


def _lin12_call(S1, xs, dinv, W1, b1r, W2):
    def body(s_ref, x_ref, d_ref, w1_ref, b_ref, w2_ref, o_ref):
        d = d_ref[...]
        t = s_ref[0] + s_ref[1] + x_ref[...]
        h = jnp.dot(t, w1_ref[...], preferred_element_type=jnp.float32)
        u = jnp.maximum(d * h + b_ref[...], 0.0)
        o_ref[...] = d * jnp.dot(u, w2_ref[...],
                                 preferred_element_type=jnp.float32)

    return pl.pallas_call(
        body,
        grid=(N // _BR,),
        in_specs=[
            pl.BlockSpec((NC, _BR, F), lambda i: (0, i, 0)),
            pl.BlockSpec((_BR, F), lambda i: (i, 0)),
            pl.BlockSpec((_BR, 1), lambda i: (i, 0)),
            pl.BlockSpec((128, 256), lambda i: (0, 0)),
            pl.BlockSpec((1, 256), lambda i: (0, 0)),
            pl.BlockSpec((256, F), lambda i: (0, 0)),
        ],
        out_specs=pl.BlockSpec((_BR, F), lambda i: (i, 0)),
        out_shape=jax.ShapeDtypeStruct((N, F), jnp.float32),
    )(S1, xs, dinv, W1, b1r, W2)


def _lin_mid_call(S, g, dinv, br, W):
    def body(s_ref, g_ref, d_ref, b_ref, w_ref, o_ref):
        d = d_ref[...]
        u = jnp.maximum(d * (s_ref[0] + s_ref[1] + g_ref[...]) + b_ref[...],
                        0.0)
        o_ref[...] = d * jnp.dot(u, w_ref[...],
                                 preferred_element_type=jnp.float32)

    return pl.pallas_call(
        body,
        grid=(N // _BR,),
        in_specs=[
            pl.BlockSpec((NC, _BR, F), lambda i: (0, i, 0)),
            pl.BlockSpec((_BR, F), lambda i: (i, 0)),
            pl.BlockSpec((_BR, 1), lambda i: (i, 0)),
            pl.BlockSpec((1, F), lambda i: (0, 0)),
            pl.BlockSpec((F, F), lambda i: (0, 0)),
        ],
        out_specs=pl.BlockSpec((_BR, F), lambda i: (i, 0)),
        out_shape=jax.ShapeDtypeStruct((N, F), jnp.float32),
    )(S, g, dinv, br, W)


def _head_call(S, g, dinv, b5r, Wm1p, bm1p, Wm2p, bm2p):
    def body(s_ref, g_ref, d_ref, b_ref, w1_ref, c1_ref, w2_ref, c2_ref,
             o_ref):
        d = d_ref[...]
        u = jnp.maximum(d * (s_ref[0] + s_ref[1] + g_ref[...]) + b_ref[...],
                        0.0)
        h2 = jnp.maximum(
            jnp.dot(u, w1_ref[...], preferred_element_type=jnp.float32)
            + c1_ref[...], 0.0)
        o_ref[...] = jnp.dot(
            h2, w2_ref[...], preferred_element_type=jnp.float32) + c2_ref[...]

    return pl.pallas_call(
        body,
        grid=(N // _BR,),
        in_specs=[
            pl.BlockSpec((NC, _BR, F), lambda i: (0, i, 0)),
            pl.BlockSpec((_BR, F), lambda i: (i, 0)),
            pl.BlockSpec((_BR, 1), lambda i: (i, 0)),
            pl.BlockSpec((1, F), lambda i: (0, 0)),
            pl.BlockSpec((F, F), lambda i: (0, 0)),
            pl.BlockSpec((1, F), lambda i: (0, 0)),
            pl.BlockSpec((F, F), lambda i: (0, 0)),
            pl.BlockSpec((1, F), lambda i: (0, 0)),
        ],
        out_specs=pl.BlockSpec((_BR, F), lambda i: (i, 0)),
        out_shape=jax.ShapeDtypeStruct((N, F), jnp.float32),
    )(S, g, dinv, b5r, Wm1p, bm1p, Wm2p, bm2p)


def kernel(x, edge_index, batch, W1, b1, W2, b2, W3, b3, W4, b4, W5, b5,
           Wm1, bm1, Wm2, bm2):
    del batch
    src2d = edge_index[0].reshape(NCH, CH)
    dst2d = edge_index[1].reshape(NCH, CH)

    partials = _degree_call(dst2d)
    dinv, xs = _dinv_call(partials, x)

    S1 = _prop_edge(src2d, dst2d, xs)
    g = _lin12_call(S1, xs, dinv, W1, b1.reshape(1, 256), W2)
    for (bl, Wn) in ((b2, W3), (b3, W4), (b4, W5)):
        S = _prop_edge(src2d, dst2d, g)
        g = _lin_mid_call(S, g, dinv, bl.reshape(1, F), Wn)

    S5 = _prop_edge(src2d, dst2d, g)

    hm = Wm1.shape[1]
    Wm1p = jnp.pad(Wm1, ((0, 0), (0, F - hm)))
    bm1p = jnp.pad(bm1, (0, F - hm)).reshape(1, F)
    Wm2p = jnp.pad(Wm2, ((0, F - hm), (0, F - Wm2.shape[1])))
    bm2p = jnp.pad(bm2, (0, F - Wm2.shape[1])).reshape(1, F)

    out = _head_call(S5, g, dinv, b5.reshape(1, F), Wm1p, bm1p, Wm2p, bm2p)
    return out[:, :Wm2.shape[1]]


# speedup vs baseline: 19.8576x; 1.0221x over previous
import functools

import jax
import jax.numpy as jnp
from jax import lax
from jax.experimental import pallas as pl
from jax.experimental.pallas import tpu as pltpu
from jax.experimental.pallas import tpu_sc as plsc

N = 10000
E = 320000
NC = 2
NS = 16
CH = 128
NCH = E // CH
RPB = 624
F = 128


def _mesh():
    return plsc.VectorSubcoreMesh(core_axis_name="c", subcore_axis_name="s")


def _degree_call(dst2d):
    @functools.partial(
        pl.kernel,
        mesh=_mesh(),
        out_type=jax.ShapeDtypeStruct((NC * NS, N), jnp.float32),
        compiler_params=pltpu.CompilerParams(needs_layout_passes=False),
        scratch_types=[
            pltpu.VMEM((N,), jnp.float32),
            pltpu.VMEM((2, CH), jnp.int32),
            pltpu.SemaphoreType.DMA((2,)),
        ],
    )
    def k(dst_hbm, out_hbm, hist_v, idx_v, isem):
        c = lax.axis_index("c")
        s = lax.axis_index("s")
        tid = s * NC + c
        z16 = jnp.zeros((16,), jnp.float32)

        def zero_body(i, carry):
            hist_v[pl.ds(i * 16, 16)] = z16
            return carry

        lax.fori_loop(0, N // 16, zero_body, 0)

        ones = jnp.ones((16,), jnp.float32)
        stride = NC * NS
        nloop = (NCH + stride - 1) // stride

        for b in range(2):
            @pl.when(b * stride + tid < NCH)
            def _(b=b):
                pltpu.async_copy(dst_hbm.at[b * stride + tid], idx_v.at[b],
                                 isem.at[b])

        def chunk_body(kk, carry):
            for b in range(2):
                j = (kk * 2 + b) * stride + tid

                @pl.when(j < NCH)
                def _(b=b, j=j):
                    pltpu.make_async_copy(dst_hbm.at[j], idx_v.at[b],
                                          isem.at[b]).wait()
                    for i in range(CH // 16):
                        idx = idx_v[b, pl.ds(i * 16, 16)]
                        plsc.addupdate_scatter(hist_v, [idx], ones)
                    jn = j + 2 * stride

                    @pl.when(jn < NCH)
                    def _():
                        pltpu.async_copy(dst_hbm.at[jn], idx_v.at[b],
                                         isem.at[b])

            return carry

        lax.fori_loop(0, (nloop + 1) // 2, chunk_body, 0)
        pltpu.sync_copy(hist_v, out_hbm.at[tid])

    return k(dst2d)


def _make_prop():
    R = 3

    @functools.partial(
        pl.kernel,
        mesh=_mesh(),
        out_type=jax.ShapeDtypeStruct((NC, N, F), jnp.float32),
        scratch_types=[
            pltpu.VMEM((R, CH), jnp.int32),
            pltpu.VMEM((R, CH), jnp.int32),
            pltpu.VMEM((R, CH, F), jnp.float32),
            pltpu.VMEM_SHARED((N, F), jnp.float32),
            pltpu.SemaphoreType.DMA((R,)),
            pltpu.SemaphoreType.DMA((R,)),
            pltpu.SemaphoreType.DMA,
        ],
    )
    def k(src_hbm, dst_hbm, g_hbm, out_hbm, src_v, dst_v, rows_v,
          slab, gsem, ssem, isem):
        c = lax.axis_index("c")
        s = lax.axis_index("s")
        z16 = jnp.zeros((16,), jnp.float32)

        def zb(i, carry):
            rows_v[0, i // (F // 16), pl.ds((i % (F // 16)) * 16, 16)] = z16
            return carry

        lax.fori_loop(0, 8 * (F // 16), zb, 0)
        zrows = rows_v.at[0, pl.ds(0, 8)]

        def zcopy(q, carry):
            pltpu.sync_copy(zrows, slab.at[pl.ds(s * RPB + q * 8, 8)])
            return carry

        lax.fori_loop(0, RPB // 8, zcopy, 0)

        @pl.when(s == 0)
        def _zero_tail():
            pltpu.sync_copy(zrows, slab.at[pl.ds(NS * RPB, 8)])
            pltpu.sync_copy(zrows, slab.at[pl.ds(NS * RPB + 8, 8)])

        plsc.subcore_barrier()

        stride = NC * NS
        base = s * NC + c
        nloop = (NCH + stride - 1) // stride

        def body(kk, carry):
            js = [(kk * R + b) * stride + base for b in range(R)]

            for b in range(R):
                @pl.when(js[b] - R * stride >= 0)
                def _(b=b):
                    pltpu.make_async_copy(rows_v.at[b],
                                          slab.at[dst_v.at[b]],
                                          ssem.at[b]).wait()

            for b in range(R):
                @pl.when(js[b] < NCH)
                def _(b=b):
                    pltpu.async_copy(src_hbm.at[js[b]], src_v.at[b], isem)
                    pltpu.async_copy(dst_hbm.at[js[b]], dst_v.at[b], isem)

            for b in range(R):
                @pl.when(js[b] < NCH)
                def _(b=b):
                    pltpu.make_async_copy(src_hbm.at[js[b]], src_v.at[b],
                                          isem).wait()
                    pltpu.make_async_copy(dst_hbm.at[js[b]], dst_v.at[b],
                                          isem).wait()
                    pltpu.async_copy(g_hbm.at[src_v.at[b]], rows_v.at[b],
                                     gsem.at[b])

            for b in range(R):
                @pl.when(js[b] < NCH)
                def _(b=b):
                    pltpu.make_async_copy(g_hbm.at[src_v.at[b]],
                                          rows_v.at[b], gsem.at[b]).wait()
                    pltpu.async_copy(rows_v.at[b], slab.at[dst_v.at[b]],
                                     ssem.at[b], add=True)

            return carry

        nbody = (nloop + R - 1) // R
        lax.fori_loop(0, nbody, body, 0)
        for b in range(R):
            @pl.when((nbody - 1) * R * stride + b * stride + base < NCH)
            def _(b=b):
                pltpu.make_async_copy(rows_v.at[b], slab.at[dst_v.at[b]],
                                      ssem.at[b]).wait()
        plsc.subcore_barrier()
        pltpu.sync_copy(slab.at[pl.ds(s * RPB, RPB)],
                        out_hbm.at[c, pl.ds(s * RPB, RPB)])

        @pl.when(s == 0)
        def _drain_tail():
            pltpu.sync_copy(slab.at[pl.ds(NS * RPB, N - NS * RPB)],
                            out_hbm.at[c, pl.ds(NS * RPB, N - NS * RPB)])

    return k


_PROP_CACHE = {}


def _prop_edge(src2d, dst2d, g):
    if 0 not in _PROP_CACHE:
        _PROP_CACHE[0] = _make_prop()
    return _PROP_CACHE[0](src2d, dst2d, g)


_BR = 2000


def _dinv_call(partials, x):
    def body(p_ref, x_ref, d_ref, xs_ref):
        deg = jnp.sum(p_ref[...], axis=0) + 1.0
        d = lax.rsqrt(deg)[:, None]
        d_ref[...] = d
        xs_ref[...] = d * x_ref[...]

    return pl.pallas_call(
        body,
        out_shape=(jax.ShapeDtypeStruct((N, 1), jnp.float32),
                   jax.ShapeDtypeStruct((N, 128), jnp.float32)),
    )(partials, x)


def _lin12_call(S1, xs, dinv, W1, b1r, W2):
    def body(s_ref, x_ref, d_ref, w1_ref, b_ref, w2_ref, o_ref):
        d = d_ref[...]
        t = s_ref[0] + s_ref[1] + x_ref[...]
        h = jnp.dot(t, w1_ref[...], preferred_element_type=jnp.float32)
        u = jnp.maximum(d * h + b_ref[...], 0.0)
        o_ref[...] = d * jnp.dot(u, w2_ref[...],
                                 preferred_element_type=jnp.float32)

    return pl.pallas_call(
        body,
        grid=(N // _BR,),
        in_specs=[
            pl.BlockSpec((NC, _BR, F), lambda i: (0, i, 0)),
            pl.BlockSpec((_BR, F), lambda i: (i, 0)),
            pl.BlockSpec((_BR, 1), lambda i: (i, 0)),
            pl.BlockSpec((128, 256), lambda i: (0, 0)),
            pl.BlockSpec((1, 256), lambda i: (0, 0)),
            pl.BlockSpec((256, F), lambda i: (0, 0)),
        ],
        out_specs=pl.BlockSpec((_BR, F), lambda i: (i, 0)),
        out_shape=jax.ShapeDtypeStruct((N, F), jnp.float32),
    )(S1, xs, dinv, W1, b1r, W2)


def _lin_mid_call(S, g, dinv, br, W):
    def body(s_ref, g_ref, d_ref, b_ref, w_ref, o_ref):
        d = d_ref[...]
        u = jnp.maximum(d * (s_ref[0] + s_ref[1] + g_ref[...]) + b_ref[...],
                        0.0)
        o_ref[...] = d * jnp.dot(u, w_ref[...],
                                 preferred_element_type=jnp.float32)

    return pl.pallas_call(
        body,
        grid=(N // _BR,),
        in_specs=[
            pl.BlockSpec((NC, _BR, F), lambda i: (0, i, 0)),
            pl.BlockSpec((_BR, F), lambda i: (i, 0)),
            pl.BlockSpec((_BR, 1), lambda i: (i, 0)),
            pl.BlockSpec((1, F), lambda i: (0, 0)),
            pl.BlockSpec((F, F), lambda i: (0, 0)),
        ],
        out_specs=pl.BlockSpec((_BR, F), lambda i: (i, 0)),
        out_shape=jax.ShapeDtypeStruct((N, F), jnp.float32),
    )(S, g, dinv, br, W)


def _head_call(S, g, dinv, b5r, Wm1p, bm1p, Wm2p, bm2p):
    def body(s_ref, g_ref, d_ref, b_ref, w1_ref, c1_ref, w2_ref, c2_ref,
             o_ref):
        d = d_ref[...]
        u = jnp.maximum(d * (s_ref[0] + s_ref[1] + g_ref[...]) + b_ref[...],
                        0.0)
        h2 = jnp.maximum(
            jnp.dot(u, w1_ref[...], preferred_element_type=jnp.float32)
            + c1_ref[...], 0.0)
        o_ref[...] = jnp.dot(
            h2, w2_ref[...], preferred_element_type=jnp.float32) + c2_ref[...]

    return pl.pallas_call(
        body,
        grid=(N // _BR,),
        in_specs=[
            pl.BlockSpec((NC, _BR, F), lambda i: (0, i, 0)),
            pl.BlockSpec((_BR, F), lambda i: (i, 0)),
            pl.BlockSpec((_BR, 1), lambda i: (i, 0)),
            pl.BlockSpec((1, F), lambda i: (0, 0)),
            pl.BlockSpec((F, F), lambda i: (0, 0)),
            pl.BlockSpec((1, F), lambda i: (0, 0)),
            pl.BlockSpec((F, F), lambda i: (0, 0)),
            pl.BlockSpec((1, F), lambda i: (0, 0)),
        ],
        out_specs=pl.BlockSpec((_BR, F), lambda i: (i, 0)),
        out_shape=jax.ShapeDtypeStruct((N, F), jnp.float32),
    )(S, g, dinv, b5r, Wm1p, bm1p, Wm2p, bm2p)


def kernel(x, edge_index, batch, W1, b1, W2, b2, W3, b3, W4, b4, W5, b5,
           Wm1, bm1, Wm2, bm2):
    del batch
    src2d = edge_index[0].reshape(NCH, CH)
    dst2d = edge_index[1].reshape(NCH, CH)

    partials = _degree_call(dst2d)
    dinv, xs = _dinv_call(partials, x)

    S1 = _prop_edge(src2d, dst2d, xs)
    g = _lin12_call(S1, xs, dinv, W1, b1.reshape(1, 256), W2)
    for (bl, Wn) in ((b2, W3), (b3, W4), (b4, W5)):
        S = _prop_edge(src2d, dst2d, g)
        g = _lin_mid_call(S, g, dinv, bl.reshape(1, F), Wn)

    S5 = _prop_edge(src2d, dst2d, g)

    hm = Wm1.shape[1]
    Wm1p = jnp.pad(Wm1, ((0, 0), (0, F - hm)))
    bm1p = jnp.pad(bm1, (0, F - hm)).reshape(1, F)
    Wm2p = jnp.pad(Wm2, ((0, F - hm), (0, F - Wm2.shape[1])))
    bm2p = jnp.pad(bm2, (0, F - Wm2.shape[1])).reshape(1, F)

    out = _head_call(S5, g, dinv, b5.reshape(1, F), Wm1p, bm1p, Wm2p, bm2p)
    return out[:, :Wm2.shape[1]]


# speedup vs baseline: 20.3263x; 1.0236x over previous
import functools

import jax
import jax.numpy as jnp
from jax import lax
from jax.experimental import pallas as pl
from jax.experimental.pallas import tpu as pltpu
from jax.experimental.pallas import tpu_sc as plsc

N = 10000
E = 320000
NC = 2
NS = 16
CH = 128
NCH = E // CH
RPB = 624
F = 128


def _mesh():
    return plsc.VectorSubcoreMesh(core_axis_name="c", subcore_axis_name="s")


def _degree_call(dst2d):
    @functools.partial(
        pl.kernel,
        mesh=_mesh(),
        out_type=jax.ShapeDtypeStruct((NC * NS, N), jnp.float32),
        compiler_params=pltpu.CompilerParams(needs_layout_passes=False),
        scratch_types=[
            pltpu.VMEM((N,), jnp.float32),
            pltpu.VMEM((2, CH), jnp.int32),
            pltpu.SemaphoreType.DMA((2,)),
        ],
    )
    def k(dst_hbm, out_hbm, hist_v, idx_v, isem):
        c = lax.axis_index("c")
        s = lax.axis_index("s")
        tid = s * NC + c
        z16 = jnp.zeros((16,), jnp.float32)

        def zero_body(i, carry):
            hist_v[pl.ds(i * 16, 16)] = z16
            return carry

        lax.fori_loop(0, N // 16, zero_body, 0)

        ones = jnp.ones((16,), jnp.float32)
        stride = NC * NS
        nloop = (NCH + stride - 1) // stride

        for b in range(2):
            @pl.when(b * stride + tid < NCH)
            def _(b=b):
                pltpu.async_copy(dst_hbm.at[b * stride + tid], idx_v.at[b],
                                 isem.at[b])

        def chunk_body(kk, carry):
            for b in range(2):
                j = (kk * 2 + b) * stride + tid

                @pl.when(j < NCH)
                def _(b=b, j=j):
                    pltpu.make_async_copy(dst_hbm.at[j], idx_v.at[b],
                                          isem.at[b]).wait()
                    for i in range(CH // 16):
                        idx = idx_v[b, pl.ds(i * 16, 16)]
                        plsc.addupdate_scatter(hist_v, [idx], ones)
                    jn = j + 2 * stride

                    @pl.when(jn < NCH)
                    def _():
                        pltpu.async_copy(dst_hbm.at[jn], idx_v.at[b],
                                         isem.at[b])

            return carry

        lax.fori_loop(0, (nloop + 1) // 2, chunk_body, 0)
        pltpu.sync_copy(hist_v, out_hbm.at[tid])

    return k(dst2d)


def _make_prop():
    R = 3

    @functools.partial(
        pl.kernel,
        mesh=_mesh(),
        out_type=jax.ShapeDtypeStruct((NC, N, F), jnp.float32),
        scratch_types=[
            pltpu.VMEM((R, CH), jnp.int32),
            pltpu.VMEM((R, 2, CH // 2), jnp.int32),
            pltpu.VMEM((R, CH, F), jnp.float32),
            pltpu.VMEM_SHARED((N, F), jnp.float32),
            pltpu.SemaphoreType.DMA((R,)),
            pltpu.SemaphoreType.DMA((R,)),
            pltpu.SemaphoreType.DMA,
        ],
    )
    def k(src_hbm, dst_hbm, g_hbm, out_hbm, src_v, dst_v, rows_v,
          slab, gsem, ssem, isem):
        c = lax.axis_index("c")
        s = lax.axis_index("s")
        z16 = jnp.zeros((16,), jnp.float32)

        def zb(i, carry):
            rows_v[0, i // (F // 16), pl.ds((i % (F // 16)) * 16, 16)] = z16
            return carry

        lax.fori_loop(0, 8 * (F // 16), zb, 0)
        zrows = rows_v.at[0, pl.ds(0, 8)]

        def zcopy(q, carry):
            pltpu.sync_copy(zrows, slab.at[pl.ds(s * RPB + q * 8, 8)])
            return carry

        lax.fori_loop(0, RPB // 8, zcopy, 0)

        @pl.when(s == 0)
        def _zero_tail():
            pltpu.sync_copy(zrows, slab.at[pl.ds(NS * RPB, 8)])
            pltpu.sync_copy(zrows, slab.at[pl.ds(NS * RPB + 8, 8)])

        plsc.subcore_barrier()

        stride = NC * NS
        base = s * NC + c
        nloop = (NCH + stride - 1) // stride

        def body(kk, carry):
            js = [(kk * R + b) * stride + base for b in range(R)]

            for b in range(R):
                @pl.when(js[b] - R * stride >= 0)
                def _(b=b):
                    for h in range(2):
                        pltpu.make_async_copy(
                            rows_v.at[b, pl.ds(h * (CH // 2), CH // 2)],
                            slab.at[dst_v.at[b, h]], ssem.at[b]).wait()

            for b in range(R):
                @pl.when(js[b] < NCH)
                def _(b=b):
                    pltpu.async_copy(src_hbm.at[js[b]], src_v.at[b], isem)
                    pltpu.async_copy(dst_hbm.at[js[b]], dst_v.at[b], isem)

            for b in range(R):
                @pl.when(js[b] < NCH)
                def _(b=b):
                    pltpu.make_async_copy(src_hbm.at[js[b]], src_v.at[b],
                                          isem).wait()
                    pltpu.make_async_copy(dst_hbm.at[js[b]], dst_v.at[b],
                                          isem).wait()
                    for h in range(2):
                        pltpu.async_copy(
                            g_hbm.at[src_v.at[b, pl.ds(h * (CH // 2),
                                                       CH // 2)]],
                            rows_v.at[b, pl.ds(h * (CH // 2), CH // 2)],
                            gsem.at[b])

            for b in range(R):
                @pl.when(js[b] < NCH)
                def _(b=b):
                    for h in range(2):
                        sl = pl.ds(h * (CH // 2), CH // 2)
                        pltpu.make_async_copy(
                            g_hbm.at[src_v.at[b, sl]], rows_v.at[b, sl],
                            gsem.at[b]).wait()
                        pltpu.async_copy(rows_v.at[b, sl],
                                         slab.at[dst_v.at[b, h]],
                                         ssem.at[b], add=True)

            return carry

        nbody = (nloop + R - 1) // R
        lax.fori_loop(0, nbody, body, 0)
        for b in range(R):
            @pl.when((nbody - 1) * R * stride + b * stride + base < NCH)
            def _(b=b):
                for h in range(2):
                    pltpu.make_async_copy(
                        rows_v.at[b, pl.ds(h * (CH // 2), CH // 2)],
                        slab.at[dst_v.at[b, h]], ssem.at[b]).wait()
        plsc.subcore_barrier()
        pltpu.sync_copy(slab.at[pl.ds(s * RPB, RPB)],
                        out_hbm.at[c, pl.ds(s * RPB, RPB)])

        @pl.when(s == 0)
        def _drain_tail():
            pltpu.sync_copy(slab.at[pl.ds(NS * RPB, N - NS * RPB)],
                            out_hbm.at[c, pl.ds(NS * RPB, N - NS * RPB)])

    return k


_PROP_CACHE = {}


def _prop_edge(src2d, dst2d, g):
    if 0 not in _PROP_CACHE:
        _PROP_CACHE[0] = _make_prop()
    return _PROP_CACHE[0](src2d, dst2d, g)


_BR = 2000


def _dinv_call(partials, x):
    def body(p_ref, x_ref, d_ref, xs_ref):
        deg = jnp.sum(p_ref[...], axis=0) + 1.0
        d = lax.rsqrt(deg)[:, None]
        d_ref[...] = d
        xs_ref[...] = d * x_ref[...]

    return pl.pallas_call(
        body,
        out_shape=(jax.ShapeDtypeStruct((N, 1), jnp.float32),
                   jax.ShapeDtypeStruct((N, 128), jnp.float32)),
    )(partials, x)


def _lin12_call(S1, xs, dinv, W1, b1r, W2):
    def body(s_ref, x_ref, d_ref, w1_ref, b_ref, w2_ref, o_ref):
        d = d_ref[...]
        t = s_ref[0] + s_ref[1] + x_ref[...]
        h = jnp.dot(t, w1_ref[...], preferred_element_type=jnp.float32)
        u = jnp.maximum(d * h + b_ref[...], 0.0)
        o_ref[...] = d * jnp.dot(u, w2_ref[...],
                                 preferred_element_type=jnp.float32)

    return pl.pallas_call(
        body,
        grid=(N // _BR,),
        in_specs=[
            pl.BlockSpec((NC, _BR, F), lambda i: (0, i, 0)),
            pl.BlockSpec((_BR, F), lambda i: (i, 0)),
            pl.BlockSpec((_BR, 1), lambda i: (i, 0)),
            pl.BlockSpec((128, 256), lambda i: (0, 0)),
            pl.BlockSpec((1, 256), lambda i: (0, 0)),
            pl.BlockSpec((256, F), lambda i: (0, 0)),
        ],
        out_specs=pl.BlockSpec((_BR, F), lambda i: (i, 0)),
        out_shape=jax.ShapeDtypeStruct((N, F), jnp.float32),
    )(S1, xs, dinv, W1, b1r, W2)


def _lin_mid_call(S, g, dinv, br, W):
    def body(s_ref, g_ref, d_ref, b_ref, w_ref, o_ref):
        d = d_ref[...]
        u = jnp.maximum(d * (s_ref[0] + s_ref[1] + g_ref[...]) + b_ref[...],
                        0.0)
        o_ref[...] = d * jnp.dot(u, w_ref[...],
                                 preferred_element_type=jnp.float32)

    return pl.pallas_call(
        body,
        grid=(N // _BR,),
        in_specs=[
            pl.BlockSpec((NC, _BR, F), lambda i: (0, i, 0)),
            pl.BlockSpec((_BR, F), lambda i: (i, 0)),
            pl.BlockSpec((_BR, 1), lambda i: (i, 0)),
            pl.BlockSpec((1, F), lambda i: (0, 0)),
            pl.BlockSpec((F, F), lambda i: (0, 0)),
        ],
        out_specs=pl.BlockSpec((_BR, F), lambda i: (i, 0)),
        out_shape=jax.ShapeDtypeStruct((N, F), jnp.float32),
    )(S, g, dinv, br, W)


def _head_call(S, g, dinv, b5r, Wm1p, bm1p, Wm2p, bm2p):
    def body(s_ref, g_ref, d_ref, b_ref, w1_ref, c1_ref, w2_ref, c2_ref,
             o_ref):
        d = d_ref[...]
        u = jnp.maximum(d * (s_ref[0] + s_ref[1] + g_ref[...]) + b_ref[...],
                        0.0)
        h2 = jnp.maximum(
            jnp.dot(u, w1_ref[...], preferred_element_type=jnp.float32)
            + c1_ref[...], 0.0)
        o_ref[...] = jnp.dot(
            h2, w2_ref[...], preferred_element_type=jnp.float32) + c2_ref[...]

    return pl.pallas_call(
        body,
        grid=(N // _BR,),
        in_specs=[
            pl.BlockSpec((NC, _BR, F), lambda i: (0, i, 0)),
            pl.BlockSpec((_BR, F), lambda i: (i, 0)),
            pl.BlockSpec((_BR, 1), lambda i: (i, 0)),
            pl.BlockSpec((1, F), lambda i: (0, 0)),
            pl.BlockSpec((F, F), lambda i: (0, 0)),
            pl.BlockSpec((1, F), lambda i: (0, 0)),
            pl.BlockSpec((F, F), lambda i: (0, 0)),
            pl.BlockSpec((1, F), lambda i: (0, 0)),
        ],
        out_specs=pl.BlockSpec((_BR, F), lambda i: (i, 0)),
        out_shape=jax.ShapeDtypeStruct((N, F), jnp.float32),
    )(S, g, dinv, b5r, Wm1p, bm1p, Wm2p, bm2p)


def kernel(x, edge_index, batch, W1, b1, W2, b2, W3, b3, W4, b4, W5, b5,
           Wm1, bm1, Wm2, bm2):
    del batch
    src2d = edge_index[0].reshape(NCH, CH)
    dst2d = edge_index[1].reshape(NCH, CH)
    dst3d = edge_index[1].reshape(NCH, 2, CH // 2)

    partials = _degree_call(dst2d)
    dinv, xs = _dinv_call(partials, x)

    S1 = _prop_edge(src2d, dst3d, xs)
    g = _lin12_call(S1, xs, dinv, W1, b1.reshape(1, 256), W2)
    for (bl, Wn) in ((b2, W3), (b3, W4), (b4, W5)):
        S = _prop_edge(src2d, dst3d, g)
        g = _lin_mid_call(S, g, dinv, bl.reshape(1, F), Wn)

    S5 = _prop_edge(src2d, dst3d, g)

    hm = Wm1.shape[1]
    Wm1p = jnp.pad(Wm1, ((0, 0), (0, F - hm)))
    bm1p = jnp.pad(bm1, (0, F - hm)).reshape(1, F)
    Wm2p = jnp.pad(Wm2, ((0, F - hm), (0, F - Wm2.shape[1])))
    bm2p = jnp.pad(bm2, (0, F - Wm2.shape[1])).reshape(1, F)

    out = _head_call(S5, g, dinv, b5.reshape(1, F), Wm1p, bm1p, Wm2p, bm2p)
    return out[:, :Wm2.shape[1]]
